# baseline (speedup 1.0000x reference)
import functools

import jax
import jax.numpy as jnp
from jax import lax
from jax.experimental import pallas as pl
from jax.experimental.pallas import tpu as pltpu
from jax.experimental.pallas import tpu_sc as plsc

CHUNK = 128
GBUF = 8
DEPTH = 4
PADW = 64
HPACK = 4
TC_COLS = 3968


def _tc_pad(table_t):
    d, v = table_t.shape

    def body(src_ref, dst_ref):
        t = src_ref[...].T
        dst_ref[...] = jnp.concatenate(
            [t, jnp.zeros((TC_COLS, PADW - d), jnp.float32)], axis=1
        )

    return pl.pallas_call(
        body,
        grid=((v + TC_COLS - 1) // TC_COLS,),
        in_specs=[pl.BlockSpec((d, TC_COLS), lambda j: (0, j))],
        out_specs=pl.BlockSpec((TC_COLS, PADW), lambda j: (j, 0)),
        out_shape=jax.ShapeDtypeStruct((v, PADW), jnp.float32),
    )(table_t)


def _tc_transpose(inter, batch, hist, emb_dim):
    n_grp = (hist + HPACK - 1) // HPACK

    def body(src_ref, dst_ref):
        x = src_ref[...]
        for r in range(HPACK):
            dst_ref[r] = x[:, r * emb_dim : (r + 1) * emb_dim].T

    return pl.pallas_call(
        body,
        grid=(n_grp,),
        in_specs=[pl.BlockSpec((batch, HPACK * emb_dim), lambda g: (g, 0))],
        out_specs=pl.BlockSpec((HPACK, emb_dim, batch), lambda g: (g, 0, 0)),
        out_shape=jax.ShapeDtypeStruct((hist, emb_dim, batch), jnp.float32),
    )(inter)


@functools.cache
def _build(batch: int, hist: int, emb_dim: int, vocab: int):
    info = plsc.get_sparse_core_info()
    nc, ns = info.num_cores, info.num_subcores
    nw = nc * ns
    b_per_w = batch // nw
    assert batch % (nw * CHUNK) == 0
    n_sub = b_per_w // CHUNK
    n_chunks = hist * n_sub
    n_grp = (hist + HPACK - 1) // HPACK

    mesh = plsc.VectorSubcoreMesh(core_axis_name="c", subcore_axis_name="s")

    def body(idx_hbm, tab_hbm, inter_hbm, idx_v, buf_v, gsem, wsem):
        wid = lax.axis_index("s") * nc + lax.axis_index("c")
        b0 = wid * b_per_w
        pltpu.sync_copy(idx_hbm.at[:, pl.ds(b0, b_per_w)], idx_v)

        def gather(j, gb):
            h = j // n_sub
            c = lax.rem(j, n_sub)
            return pltpu.make_async_copy(
                tab_hbm.at[idx_v.at[h, pl.ds(c * CHUNK, CHUNK)]],
                buf_v.at[gb],
                gsem.at[gb],
            )

        def write(j, gb):
            h = j // n_sub
            c = lax.rem(j, n_sub)
            row0 = (h // HPACK) * batch + b0 + c * CHUNK
            col0 = lax.rem(h, HPACK) * emb_dim
            return pltpu.make_async_copy(
                buf_v.at[gb],
                inter_hbm.at[pl.ds(row0, CHUNK), pl.ds(col0, emb_dim)],
                wsem.at[gb],
            )

        for j in range(DEPTH):
            gather(j, j).start()

        def step(j, carry):
            gb = lax.rem(j, jnp.int32(GBUF))
            gbn = lax.rem(j + DEPTH, jnp.int32(GBUF))

            @pl.when(j + DEPTH < n_chunks)
            def _refill():
                @pl.when(j >= GBUF - DEPTH)
                def _drain():
                    write(j + DEPTH - GBUF, gbn).wait()

                gather(j + DEPTH, gbn).start()

            gather(j, gb).wait()
            write(j, gb).start()
            return carry

        lax.fori_loop(0, n_chunks, step, 0, unroll=False)
        for j in range(n_chunks - GBUF, n_chunks):
            write(j, j % GBUF).wait()

    return pl.kernel(
        body,
        out_type=jax.ShapeDtypeStruct((n_grp * batch, HPACK * emb_dim), jnp.float32),
        mesh=mesh,
        compiler_params=pltpu.CompilerParams(
            use_tc_tiling_on_sc=False, needs_layout_passes=False
        ),
        scratch_types=[
            pltpu.VMEM((hist, b_per_w), jnp.int32),
            pltpu.VMEM((GBUF, CHUNK, emb_dim), jnp.float32),
            pltpu.SemaphoreType.DMA((GBUF,)),
            pltpu.SemaphoreType.DMA((GBUF,)),
        ],
    )


def kernel(input, table):
    batch, hist = input.shape
    vocab, emb_dim = table.shape
    k = _build(batch, hist, emb_dim, vocab)
    scale = PADW // emb_dim
    inp_t = input.T.astype(jnp.int32) * jnp.int32(scale)
    tabp = _tc_pad(table.T)
    tab2 = tabp.reshape(vocab * scale, emb_dim)
    inter = k(inp_t, tab2)
    out = _tc_transpose(inter, batch, hist, emb_dim)
    return out.transpose(2, 0, 1)


# speedup vs baseline: 1.9781x; 1.9781x over previous
import functools

import jax
import jax.numpy as jnp
from jax import lax
from jax.experimental import pallas as pl
from jax.experimental.pallas import tpu as pltpu
from jax.experimental.pallas import tpu_sc as plsc

CHUNK = 128
GBUF = 8
DEPTH = 4
PADW = 128
HPACK = 2


TC_COLS = 7936


def _tc_pad(table_t):
    d, v = table_t.shape

    def body(src_ref, dst_ref):
        t = src_ref[...].T
        dst_ref[...] = jnp.concatenate(
            [t, jnp.zeros((TC_COLS, PADW - d), jnp.float32)], axis=1
        )

    return pl.pallas_call(
        body,
        grid=((v + TC_COLS - 1) // TC_COLS,),
        in_specs=[pl.BlockSpec((d, TC_COLS), lambda j: (0, j))],
        out_specs=pl.BlockSpec((TC_COLS, PADW), lambda j: (j, 0)),
        out_shape=jax.ShapeDtypeStruct((v, PADW), jnp.float32),
    )(table_t)


def _tc_transpose(inter, batch, hist, emb_dim):
    assert hist % HPACK == 0

    def body(src_ref, dst_ref):
        x = src_ref[...]
        for r in range(HPACK):
            dst_ref[r] = x[:, r * emb_dim : (r + 1) * emb_dim].T

    return pl.pallas_call(
        body,
        grid=(hist // HPACK,),
        in_specs=[pl.BlockSpec((batch, PADW), lambda g: (g, 0))],
        out_specs=pl.BlockSpec((HPACK, emb_dim, batch), lambda g: (g, 0, 0)),
        out_shape=jax.ShapeDtypeStruct((hist, emb_dim, batch), jnp.float32),
    )(inter)


@functools.cache
def _build(batch: int, hist: int, emb_dim: int, vocab: int):
    info = plsc.get_sparse_core_info()
    nc, ns = info.num_cores, info.num_subcores
    nw = nc * ns
    b_per_w = batch // nw
    assert batch % (nw * CHUNK) == 0
    n_sub = b_per_w // CHUNK
    n_chunks = hist * n_sub

    mesh = plsc.VectorSubcoreMesh(core_axis_name="c", subcore_axis_name="s")

    def body(idx_hbm, tab_hbm, inter_hbm, idx_v, buf_v, gsem, wsem):
        wid = lax.axis_index("s") * nc + lax.axis_index("c")
        b0 = wid * b_per_w
        pltpu.sync_copy(idx_hbm.at[:, pl.ds(b0, b_per_w)], idx_v)

        def gather(j, gb):
            h = j // n_sub
            c = lax.rem(j, n_sub)
            return pltpu.make_async_copy(
                tab_hbm.at[idx_v.at[h, pl.ds(c * CHUNK, CHUNK)]],
                buf_v.at[gb],
                gsem.at[gb],
            )

        def write(j, gb):
            h = j // n_sub
            c = lax.rem(j, n_sub)
            row0 = (h // HPACK) * batch + b0 + c * CHUNK
            col0 = lax.rem(h, HPACK) * emb_dim
            return pltpu.make_async_copy(
                buf_v.at[gb],
                inter_hbm.at[pl.ds(row0, CHUNK), pl.ds(col0, emb_dim)],
                wsem.at[gb],
            )

        for j in range(DEPTH):
            gather(j, j).start()

        def step(j, carry):
            gb = lax.rem(j, jnp.int32(GBUF))
            gbn = lax.rem(j + DEPTH, jnp.int32(GBUF))

            @pl.when(j + DEPTH < n_chunks)
            def _refill():
                @pl.when(j >= GBUF - DEPTH)
                def _drain():
                    write(j + DEPTH - GBUF, gbn).wait()

                gather(j + DEPTH, gbn).start()

            gather(j, gb).wait()
            write(j, gb).start()
            return carry

        lax.fori_loop(0, n_chunks, step, 0, unroll=False)
        for j in range(n_chunks - GBUF, n_chunks):
            write(j, j % GBUF).wait()

    return pl.kernel(
        body,
        out_type=jax.ShapeDtypeStruct((hist // HPACK * batch, PADW), jnp.float32),
        mesh=mesh,
        compiler_params=pltpu.CompilerParams(
            use_tc_tiling_on_sc=False, needs_layout_passes=False
        ),
        scratch_types=[
            pltpu.VMEM((hist, b_per_w), jnp.int32),
            pltpu.VMEM((GBUF, CHUNK, emb_dim), jnp.float32),
            pltpu.SemaphoreType.DMA((GBUF,)),
            pltpu.SemaphoreType.DMA((GBUF,)),
        ],
    )


def kernel(input, table):
    batch, hist = input.shape
    vocab, emb_dim = table.shape
    k = _build(batch, hist, emb_dim, vocab)
    scale = PADW // emb_dim
    inp_t = input.T.astype(jnp.int32) * jnp.int32(scale)
    tabp = _tc_pad(table.T)
    tab4 = tabp.reshape(vocab * scale, emb_dim)
    inter = k(inp_t, tab4)
    out = _tc_transpose(inter, batch, hist, emb_dim)
    return out.transpose(2, 0, 1)


# speedup vs baseline: 2.4325x; 1.2297x over previous
import functools

import jax
import jax.numpy as jnp
from jax import lax
from jax.experimental import pallas as pl
from jax.experimental.pallas import tpu as pltpu
from jax.experimental.pallas import tpu_sc as plsc

CHUNK = 128
GBUF = 8
DEPTH = 4
PADW = 128
HPACK = 2


TC_COLS = 15872


def _tc_pad(table_t):
    d, v = table_t.shape

    def body(src_ref, dst_ref):
        t = src_ref[...].T
        dst_ref[...] = jnp.concatenate(
            [t, jnp.zeros((TC_COLS, PADW - d), jnp.float32)], axis=1
        )

    return pl.pallas_call(
        body,
        grid=((v + TC_COLS - 1) // TC_COLS,),
        in_specs=[pl.BlockSpec((d, TC_COLS), lambda j: (0, j))],
        out_specs=pl.BlockSpec((TC_COLS, PADW), lambda j: (j, 0)),
        out_shape=jax.ShapeDtypeStruct((v, PADW), jnp.float32),
    )(table_t)


def _tc_transpose(inter, batch, hist, emb_dim):
    assert hist % HPACK == 0

    def body(src_ref, dst_ref):
        x = src_ref[...]
        eye = jnp.float32(1.0) * (
            lax.broadcasted_iota(jnp.int32, (emb_dim, emb_dim), 0)
            == lax.broadcasted_iota(jnp.int32, (emb_dim, emb_dim), 1)
        )
        for r in range(HPACK):
            band = x[:, r * emb_dim : (r + 1) * emb_dim]
            dst_ref[r] = jax.lax.dot_general(
                eye, band, (((1,), (1,)), ((), ()))
            )

    return pl.pallas_call(
        body,
        grid=(hist // HPACK,),
        in_specs=[pl.BlockSpec((batch, PADW), lambda g: (g, 0))],
        out_specs=pl.BlockSpec((HPACK, emb_dim, batch), lambda g: (g, 0, 0)),
        out_shape=jax.ShapeDtypeStruct((hist, emb_dim, batch), jnp.float32),
    )(inter)


@functools.cache
def _build(batch: int, hist: int, emb_dim: int, vocab: int):
    info = plsc.get_sparse_core_info()
    nc, ns = info.num_cores, info.num_subcores
    nw = nc * ns
    b_per_w = batch // nw
    assert batch % (nw * CHUNK) == 0
    n_sub = b_per_w // CHUNK
    n_chunks = hist * n_sub

    mesh = plsc.VectorSubcoreMesh(core_axis_name="c", subcore_axis_name="s")

    def body(idx_hbm, tab_hbm, inter_hbm, idx_v, buf_v, gsem, wsem):
        wid = lax.axis_index("s") * nc + lax.axis_index("c")
        b0 = wid * b_per_w
        pltpu.sync_copy(idx_hbm.at[:, pl.ds(b0, b_per_w)], idx_v)

        def gather(j, gb):
            h = j // n_sub
            c = lax.rem(j, n_sub)
            return pltpu.make_async_copy(
                tab_hbm.at[idx_v.at[h, pl.ds(c * CHUNK, CHUNK)]],
                buf_v.at[gb],
                gsem.at[gb],
            )

        def write(j, gb):
            h = j // n_sub
            c = lax.rem(j, n_sub)
            row0 = (h // HPACK) * batch + b0 + c * CHUNK
            col0 = lax.rem(h, HPACK) * emb_dim
            return pltpu.make_async_copy(
                buf_v.at[gb],
                inter_hbm.at[pl.ds(row0, CHUNK), pl.ds(col0, emb_dim)],
                wsem.at[gb],
            )

        for j in range(DEPTH):
            gather(j, j).start()

        def step(j, carry):
            gb = lax.rem(j, jnp.int32(GBUF))
            gbn = lax.rem(j + DEPTH, jnp.int32(GBUF))

            @pl.when(j + DEPTH < n_chunks)
            def _refill():
                @pl.when(j >= GBUF - DEPTH)
                def _drain():
                    write(j + DEPTH - GBUF, gbn).wait()

                gather(j + DEPTH, gbn).start()

            gather(j, gb).wait()
            write(j, gb).start()
            return carry

        lax.fori_loop(0, n_chunks, step, 0, unroll=False)
        for j in range(n_chunks - GBUF, n_chunks):
            write(j, j % GBUF).wait()

    return pl.kernel(
        body,
        out_type=jax.ShapeDtypeStruct((hist // HPACK * batch, PADW), jnp.float32),
        mesh=mesh,
        compiler_params=pltpu.CompilerParams(
            use_tc_tiling_on_sc=False, needs_layout_passes=False
        ),
        scratch_types=[
            pltpu.VMEM((hist, b_per_w), jnp.int32),
            pltpu.VMEM((GBUF, CHUNK, emb_dim), jnp.float32),
            pltpu.SemaphoreType.DMA((GBUF,)),
            pltpu.SemaphoreType.DMA((GBUF,)),
        ],
    )


def kernel(input, table):
    batch, hist = input.shape
    vocab, emb_dim = table.shape
    k = _build(batch, hist, emb_dim, vocab)
    scale = PADW // emb_dim
    inp_t = input.T.astype(jnp.int32) * jnp.int32(scale)
    tabp = _tc_pad(table.T)
    tab4 = tabp.reshape(vocab * scale, emb_dim)
    inter = k(inp_t, tab4)
    out = _tc_transpose(inter, batch, hist, emb_dim)
    return out.transpose(2, 0, 1)
